# Initial kernel scaffold; baseline (speedup 1.0000x reference)
import jax
import jax.numpy as jnp
from jax.experimental import pallas as pl


def kernel(pred, vel, y, mass, batch):
    raise NotImplementedError("write your pallas kernel here")


# speedup vs baseline: 13.3874x; 13.3874x over previous
import functools

import jax
import jax.numpy as jnp
from jax import lax
from jax.experimental import pallas as pl
from jax.experimental.pallas import tpu as pltpu
from jax.experimental.pallas import tpu_sc as plsc

_N = 100000
_NUM_SEG = 100
_W = 0.0001
_NW = 32
_CHUNK = 3136
_NPAD = _NW * _CHUNK
_SEGP = 128
_ITERS = _CHUNK // 16


def _sc_body(pv_hbm, v_hbm, m_hbm, b_hbm, out_hbm,
             pv0, pv1, pv2, v0, v1, v2, mv, bv, a0, a1, a2, r0, r1, r2):
    wid = lax.axis_index("s") * 2 + lax.axis_index("c")
    base = wid * _CHUNK
    pltpu.sync_copy(pv_hbm.at[pl.ds(0 * _NPAD + base, _CHUNK)], pv0)
    pltpu.sync_copy(pv_hbm.at[pl.ds(1 * _NPAD + base, _CHUNK)], pv1)
    pltpu.sync_copy(pv_hbm.at[pl.ds(2 * _NPAD + base, _CHUNK)], pv2)
    pltpu.sync_copy(v_hbm.at[pl.ds(0 * _NPAD + base, _CHUNK)], v0)
    pltpu.sync_copy(v_hbm.at[pl.ds(1 * _NPAD + base, _CHUNK)], v1)
    pltpu.sync_copy(v_hbm.at[pl.ds(2 * _NPAD + base, _CHUNK)], v2)
    pltpu.sync_copy(m_hbm.at[pl.ds(base, _CHUNK)], mv)
    pltpu.sync_copy(b_hbm.at[pl.ds(base, _CHUNK)], bv)

    zeros = jnp.zeros((16,), jnp.float32)

    def zero_body(i, carry):
        a0[pl.ds(i * 16, 16)] = zeros
        a1[pl.ds(i * 16, 16)] = zeros
        a2[pl.ds(i * 16, 16)] = zeros
        return carry

    lax.fori_loop(0, 16 * _SEGP // 16, zero_body, 0)

    lane_off = lax.broadcasted_iota(jnp.int32, (16,), 0) * _SEGP

    def it(i, carry):
        s = i * 16
        idx = bv[pl.ds(s, 16)] + lane_off
        m = mv[pl.ds(s, 16)]
        plsc.addupdate_scatter(a0, [idx], m * (pv0[pl.ds(s, 16)] - v0[pl.ds(s, 16)]))
        plsc.addupdate_scatter(a1, [idx], m * (pv1[pl.ds(s, 16)] - v1[pl.ds(s, 16)]))
        plsc.addupdate_scatter(a2, [idx], m * (pv2[pl.ds(s, 16)] - v2[pl.ds(s, 16)]))
        return carry

    lax.fori_loop(0, _ITERS, it, 0)

    for a, r in ((a0, r0), (a1, r1), (a2, r2)):
        for k in range(_SEGP // 16):
            tot = a[pl.ds(k * 16, 16)]
            for lane in range(1, 16):
                tot = tot + a[pl.ds(lane * _SEGP + k * 16, 16)]
            r[pl.ds(k * 16, 16)] = tot

    obase = wid * 3 * _SEGP
    pltpu.sync_copy(r0, out_hbm.at[pl.ds(obase + 0 * _SEGP, _SEGP)])
    pltpu.sync_copy(r1, out_hbm.at[pl.ds(obase + 1 * _SEGP, _SEGP)])
    pltpu.sync_copy(r2, out_hbm.at[pl.ds(obase + 2 * _SEGP, _SEGP)])


_sc_partials = functools.partial(
    pl.kernel,
    mesh=plsc.VectorSubcoreMesh(core_axis_name="c", subcore_axis_name="s"),
    out_type=jax.ShapeDtypeStruct((_NW * 3 * _SEGP,), jnp.float32),
    compiler_params=pltpu.CompilerParams(needs_layout_passes=False),
    scratch_types=[
        pltpu.VMEM((_CHUNK,), jnp.float32),
        pltpu.VMEM((_CHUNK,), jnp.float32),
        pltpu.VMEM((_CHUNK,), jnp.float32),
        pltpu.VMEM((_CHUNK,), jnp.float32),
        pltpu.VMEM((_CHUNK,), jnp.float32),
        pltpu.VMEM((_CHUNK,), jnp.float32),
        pltpu.VMEM((_CHUNK,), jnp.float32),
        pltpu.VMEM((_CHUNK,), jnp.int32),
        pltpu.VMEM((16 * _SEGP,), jnp.float32),
        pltpu.VMEM((16 * _SEGP,), jnp.float32),
        pltpu.VMEM((16 * _SEGP,), jnp.float32),
        pltpu.VMEM((_SEGP,), jnp.float32),
        pltpu.VMEM((_SEGP,), jnp.float32),
        pltpu.VMEM((_SEGP,), jnp.float32),
    ],
)(_sc_body)


def _tc_body(p_ref, o_ref):
    x = p_ref[...]
    s = jnp.sum(x, axis=0)
    o_ref[0, 0] = jnp.sum(s * s) * (_W / (3.0 * _NUM_SEG))


_tc_finish = pl.pallas_call(
    _tc_body,
    out_shape=jax.ShapeDtypeStruct((1, 1), jnp.float32),
    out_specs=pl.BlockSpec(memory_space=pltpu.SMEM),
)


def kernel(pred, vel, y, mass, batch):
    del y
    pvT = jnp.pad(pred[:, 3:6].T, ((0, 0), (0, _NPAD - _N))).reshape(-1)
    vT = jnp.pad(vel.T, ((0, 0), (0, _NPAD - _N))).reshape(-1)
    mp = jnp.pad(mass, (0, _NPAD - _N))
    bp = jnp.pad(batch.astype(jnp.int32), (0, _NPAD - _N))
    partials = _sc_partials(pvT, vT, mp, bp)
    return _tc_finish(partials.reshape(_NW, 3, _SEGP))[0, 0]
